# baseline (speedup 1.0000x reference)
import functools

import jax
import jax.numpy as jnp
from jax import lax
from jax.experimental import pallas as pl
from jax.experimental.pallas import tpu as pltpu
from jax.experimental.pallas import tpu_sc as plsc

_N = 50000
_F = 128
_S = 9
_O = 128

_MM_BLOCK = 400
_MM_GRID = _N // _MM_BLOCK

_NC = 2
_NS = 16
_NW = _NC * _NS
_CPW = 1600
_NPAD = _NW * _CPW
_CH = 16
_NCHUNK = _CPW // _CH
_DEPTH = 4
_LANE = 16
_GROUPS = _O // _LANE


def _mm_body(x_ref, wt_ref, *y_refs):
    d = jnp.dot(x_ref[...], wt_ref[...], preferred_element_type=jnp.float32)
    for s in range(_S):
        y_refs[s][...] = d[:, s * _O:(s + 1) * _O]


def _tc_matmul(x2, wt):
    return pl.pallas_call(
        _mm_body,
        grid=(_MM_GRID,),
        in_specs=[
            pl.BlockSpec((_MM_BLOCK, _F), lambda i: (i, 0)),
            pl.BlockSpec((_F, _S * _O), lambda i: (0, 0)),
        ],
        out_specs=[pl.BlockSpec((_MM_BLOCK, _O), lambda i: (i, 0))
                   for _ in range(_S)],
        out_shape=[jax.ShapeDtypeStruct((_N, _O), jnp.float32)
                   for _ in range(_S)],
    )(x2, wt)


@functools.partial(
    pl.kernel,
    out_type=jax.ShapeDtypeStruct((_NPAD, _O), jnp.float32),
    mesh=plsc.VectorSubcoreMesh(core_axis_name="c", subcore_axis_name="s"),
    scratch_types=[
        pltpu.VMEM((_S * _CPW,), jnp.int32),
        pltpu.VMEM((_DEPTH, _CH, _O), jnp.float32),
        pltpu.VMEM((_CH, _O), jnp.float32),
        pltpu.VMEM((_CH, _O), jnp.float32),
        pltpu.VMEM((_O,), jnp.float32),
        [pltpu.SemaphoreType.DMA] * _DEPTH,
        pltpu.SemaphoreType.DMA,
        pltpu.SemaphoreType.DMA,
    ],
)
def _sc_gather_reduce(y0, y1, y2, y3, y4, y5, y6, y7, y8,
                      idx_hbm, b_hbm, out_hbm,
                      idx_v, rowsb, outb0, outb1, bias_v,
                      sems, semo0, semo1):
    ys = (y0, y1, y2, y3, y4, y5, y6, y7, y8)
    wid = lax.axis_index("s") * _NC + lax.axis_index("c")
    node_base = wid * _CPW

    for s in range(_S):
        pltpu.sync_copy(idx_hbm.at[pl.ds(s * _NPAD + node_base, _CPW)],
                        idx_v.at[pl.ds(s * _CPW, _CPW)])
    pltpu.sync_copy(b_hbm, bias_v)

    def bias_fill(slot):
        for n in range(_CH):
            for j in range(_GROUPS):
                sl = pl.ds(j * _LANE, _LANE)
                rowsb[slot, n, sl] = bias_v[sl]

    def start_gathers(g, slot):
        for s in range(_S):
            src = ys[s].at[idx_v.at[pl.ds(s * _CPW + g * _CH, _CH)]]
            pltpu.async_copy(src, rowsb.at[slot], sems[slot], add=True)

    def wait_gathers(g, slot):
        for s in range(_S):
            src = ys[s].at[idx_v.at[pl.ds(s * _CPW + g * _CH, _CH)]]
            pltpu.make_async_copy(src, rowsb.at[slot], sems[slot]).wait()

    for g0 in range(_DEPTH):
        bias_fill(g0)
        start_gathers(g0, g0)

    def out_copy(g, outb, semo):
        dst = out_hbm.at[pl.ds(node_base + g * _CH, _CH)]
        return pltpu.make_async_copy(outb, dst, semo)

    def compute(g, slot, outb):
        for n in range(_CH):
            nid = node_base + g * _CH + n
            keep = (nid != _N - 1).astype(jnp.float32)
            for j in range(_GROUPS):
                sl = pl.ds(j * _LANE, _LANE)
                v = rowsb[slot, n, sl]
                v = jnp.where(v > 0.0, v, jnp.exp(v) - 1.0)
                outb[n, sl] = v * keep
                rowsb[slot, n, sl] = bias_v[sl]

    def body(h, carry):
        for p in range(_DEPTH):
            g = _DEPTH * h + p
            outb, semo = (outb0, semo0) if p % 2 == 0 else (outb1, semo1)
            wait_gathers(g, p)

            @pl.when(g >= 2)
            def _():
                out_copy(g - 2, outb, semo).wait()

            compute(g, p, outb)
            out_copy(g, outb, semo).start()
            nxt = g + _DEPTH

            @pl.when(nxt < _NCHUNK)
            def _():
                start_gathers(nxt, p)
        return carry

    lax.fori_loop(0, _NCHUNK // _DEPTH, body, 0)
    out_copy(_NCHUNK - 2, outb0, semo0).wait()
    out_copy(_NCHUNK - 1, outb1, semo1).wait()


def kernel(x, spiral_adj, W, b):
    B, N, F = x.shape
    S = spiral_adj.shape[-1]
    O = W.shape[0]
    assert (B, N, F, S, O) == (1, _N, _F, _S, _O)

    x2 = x.reshape(N, F)
    wt = jnp.transpose(W.reshape(O, S, F), (2, 1, 0)).reshape(F, S * O)
    ys = _tc_matmul(x2, wt)

    idxT = jnp.pad(spiral_adj[0].astype(jnp.int32).T, ((0, 0), (0, _NPAD - N)))
    idx2 = idxT.reshape(-1)

    out = _sc_gather_reduce(*ys, idx2, b)
    return out[:N].reshape(B, N, O)


# speedup vs baseline: 1.5847x; 1.5847x over previous
import functools

import jax
import jax.numpy as jnp
from jax import lax
from jax.experimental import pallas as pl
from jax.experimental.pallas import tpu as pltpu
from jax.experimental.pallas import tpu_sc as plsc

_N = 50000
_F = 128
_S = 9
_O = 128

_MM_BLOCK = 1000
_MM_GRID = _N // _MM_BLOCK

_NC = 2
_NS = 16
_NW = _NC * _NS
_CPW = 1600
_NPAD = _NW * _CPW
_CH = 8
_NCHUNK = _CPW // _CH
_NREAL_LAST = (_N - 31 * _CPW) // _CH
_DEPTH = 2
_LANE = 16
_GROUPS = _O // _LANE


def _mm_body(x_ref, wt_ref, *y_refs):
    d = jnp.dot(x_ref[...], wt_ref[...], preferred_element_type=jnp.float32)
    for s in range(_S):
        y_refs[s][...] = d[:, s * _O:(s + 1) * _O]


def _tc_matmul(x2, wt):
    return pl.pallas_call(
        _mm_body,
        grid=(_MM_GRID,),
        in_specs=[
            pl.BlockSpec((_MM_BLOCK, _F), lambda i: (i, 0)),
            pl.BlockSpec((_F, _S * _O), lambda i: (0, 0)),
        ],
        out_specs=[pl.BlockSpec((_MM_BLOCK, _O), lambda i: (i, 0))
                   for _ in range(_S)],
        out_shape=[jax.ShapeDtypeStruct((_N, _O), jnp.float32)
                   for _ in range(_S)],
    )(x2, wt)


@functools.partial(
    pl.kernel,
    out_type=jax.ShapeDtypeStruct((_N, _O), jnp.float32),
    mesh=plsc.VectorSubcoreMesh(core_axis_name="c", subcore_axis_name="s"),
    scratch_types=[
        pltpu.VMEM((_S * _CPW,), jnp.int32),
        pltpu.VMEM((_DEPTH, _CH, _O), jnp.float32),
        pltpu.VMEM((_CH, _O), jnp.float32),
        pltpu.VMEM((_CH, _O), jnp.float32),
        pltpu.VMEM((_O,), jnp.float32),
        [pltpu.SemaphoreType.DMA] * _DEPTH,
        pltpu.SemaphoreType.DMA,
        pltpu.SemaphoreType.DMA,
    ],
)
def _sc_gather_reduce(y0, y1, y2, y3, y4, y5, y6, y7, y8,
                      idx_hbm, b_hbm, out_hbm,
                      idx_v, rowsb, outb0, outb1, bias_v,
                      sems, semo0, semo1):
    ys = (y0, y1, y2, y3, y4, y5, y6, y7, y8)
    wid = lax.axis_index("s") * _NC + lax.axis_index("c")
    node_base = wid * _CPW
    nreal = jnp.where(wid == _NW - 1, _NREAL_LAST, _NCHUNK)

    for s in range(_S):
        pltpu.sync_copy(idx_hbm.at[pl.ds(s * _NPAD + node_base, _CPW)],
                        idx_v.at[pl.ds(s * _CPW, _CPW)])
    pltpu.sync_copy(b_hbm, bias_v)

    def bias_fill(slot):
        for n in range(_CH):
            for j in range(_GROUPS):
                sl = pl.ds(j * _LANE, _LANE)
                rowsb[slot, n, sl] = bias_v[sl]

    def start_gathers(g, slot):
        for s in range(_S):
            src = ys[s].at[idx_v.at[pl.ds(s * _CPW + g * _CH, _CH)]]
            pltpu.async_copy(src, rowsb.at[slot], sems[slot], add=True)

    def wait_gathers(g, slot):
        for s in range(_S):
            src = ys[s].at[idx_v.at[pl.ds(s * _CPW + g * _CH, _CH)]]
            pltpu.make_async_copy(src, rowsb.at[slot], sems[slot]).wait()

    for g0 in range(_DEPTH):
        bias_fill(g0)
        start_gathers(g0, g0)

    def out_copy(g, outb, semo):
        dst = out_hbm.at[pl.ds(node_base + g * _CH, _CH)]
        return pltpu.make_async_copy(outb, dst, semo)

    def compute(g, slot, outb):
        for n in range(_CH):
            nid = node_base + g * _CH + n
            keep = (nid != _N - 1).astype(jnp.float32)
            for j in range(_GROUPS):
                sl = pl.ds(j * _LANE, _LANE)
                v = rowsb[slot, n, sl]
                v = jnp.where(v > 0.0, v, jnp.exp(v) - 1.0)
                outb[n, sl] = v * keep
                rowsb[slot, n, sl] = bias_v[sl]

    def body(h, carry):
        for p in range(_DEPTH):
            g = _DEPTH * h + p
            outb, semo = (outb0, semo0) if p % 2 == 0 else (outb1, semo1)
            wait_gathers(g, p)

            @pl.when(g >= 2)
            def _():
                out_copy(g - 2, outb, semo).wait()

            compute(g, p, outb)
            out_copy(g, outb, semo).start()
            nxt = g + _DEPTH

            @pl.when(nxt < nreal)
            def _():
                start_gathers(nxt, p)
        return carry

    lax.fori_loop(0, nreal // _DEPTH, body, 0)
    out_copy(nreal - 2, outb0, semo0).wait()
    out_copy(nreal - 1, outb1, semo1).wait()


def kernel(x, spiral_adj, W, b):
    B, N, F = x.shape
    S = spiral_adj.shape[-1]
    O = W.shape[0]
    assert (B, N, F, S, O) == (1, _N, _F, _S, _O)

    x2 = x.reshape(N, F)
    wt = jnp.transpose(W.reshape(O, S, F), (2, 1, 0)).reshape(F, S * O)
    ys = _tc_matmul(x2, wt)

    idxT = jnp.pad(spiral_adj[0].astype(jnp.int32).T, ((0, 0), (0, _NPAD - N)))
    idx2 = idxT.reshape(-1)

    out = _sc_gather_reduce(*ys, idx2, b)
    return out.reshape(B, N, O)


# speedup vs baseline: 1.6235x; 1.0245x over previous
import functools

import jax
import jax.numpy as jnp
from jax import lax
from jax.experimental import pallas as pl
from jax.experimental.pallas import tpu as pltpu
from jax.experimental.pallas import tpu_sc as plsc

_N = 50000
_F = 128
_S = 9
_O = 128

_MM_BLOCK = 2000
_MM_GRID = _N // _MM_BLOCK

_NC = 2
_NS = 16
_NW = _NC * _NS
_CPW = 1600
_NPAD = _NW * _CPW
_CH = 8
_NCHUNK = _CPW // _CH
_NREAL_LAST = (_N - 31 * _CPW) // _CH
_DEPTH = 2
_LANE = 16
_GROUPS = _O // _LANE


def _mm_body(x_ref, wt_ref, *y_refs):
    d = jnp.dot(x_ref[...], wt_ref[...], preferred_element_type=jnp.float32)
    for s in range(_S):
        y_refs[s][...] = d[:, s * _O:(s + 1) * _O]


def _tc_matmul(x2, wt):
    return pl.pallas_call(
        _mm_body,
        grid=(_MM_GRID,),
        in_specs=[
            pl.BlockSpec((_MM_BLOCK, _F), lambda i: (i, 0)),
            pl.BlockSpec((_F, _S * _O), lambda i: (0, 0)),
        ],
        out_specs=[pl.BlockSpec((_MM_BLOCK, _O), lambda i: (i, 0))
                   for _ in range(_S)],
        out_shape=[jax.ShapeDtypeStruct((_N, _O), jnp.float32)
                   for _ in range(_S)],
    )(x2, wt)


@functools.partial(
    pl.kernel,
    out_type=jax.ShapeDtypeStruct((_N, _O), jnp.float32),
    mesh=plsc.VectorSubcoreMesh(core_axis_name="c", subcore_axis_name="s"),
    scratch_types=[
        pltpu.VMEM((_S * _CPW,), jnp.int32),
        pltpu.VMEM((_DEPTH, _CH, _O), jnp.float32),
        pltpu.VMEM((_CH, _O), jnp.float32),
        pltpu.VMEM((_CH, _O), jnp.float32),
        pltpu.VMEM((_O,), jnp.float32),
        [pltpu.SemaphoreType.DMA] * _DEPTH,
        pltpu.SemaphoreType.DMA,
        pltpu.SemaphoreType.DMA,
    ],
)
def _sc_gather_reduce(y0, y1, y2, y3, y4, y5, y6, y7, y8,
                      idx_hbm, b_hbm, out_hbm,
                      idx_v, rowsb, outb0, outb1, bias_v,
                      sems, semo0, semo1):
    ys = (y0, y1, y2, y3, y4, y5, y6, y7, y8)
    wid = lax.axis_index("s") * _NC + lax.axis_index("c")
    node_base = wid * _CPW
    nreal = jnp.where(wid == _NW - 1, _NREAL_LAST, _NCHUNK)

    for s in range(_S):
        pltpu.sync_copy(idx_hbm.at[pl.ds(s * _NPAD + node_base, _CPW)],
                        idx_v.at[pl.ds(s * _CPW, _CPW)])
    pltpu.sync_copy(b_hbm, bias_v)

    def bias_fill(slot):
        for n in range(_CH):
            for j in range(_GROUPS):
                sl = pl.ds(j * _LANE, _LANE)
                rowsb[slot, n, sl] = bias_v[sl]

    def start_gathers(g, slot):
        for s in range(_S):
            src = ys[s].at[idx_v.at[pl.ds(s * _CPW + g * _CH, _CH)]]
            pltpu.async_copy(src, rowsb.at[slot], sems[slot], add=True)

    def wait_gathers(g, slot):
        for s in range(_S):
            src = ys[s].at[idx_v.at[pl.ds(s * _CPW + g * _CH, _CH)]]
            pltpu.make_async_copy(src, rowsb.at[slot], sems[slot]).wait()

    for g0 in range(_DEPTH):
        bias_fill(g0)
        start_gathers(g0, g0)

    def out_copy(g, outb, semo):
        dst = out_hbm.at[pl.ds(node_base + g * _CH, _CH)]
        return pltpu.make_async_copy(outb, dst, semo)

    def compute(g, slot, outb):
        for n in range(_CH):
            nid = node_base + g * _CH + n
            keep = (nid != _N - 1).astype(jnp.float32)
            for j in range(_GROUPS):
                sl = pl.ds(j * _LANE, _LANE)
                v = rowsb[slot, n, sl]
                v = jnp.where(v > 0.0, v, jnp.exp(v) - 1.0)
                outb[n, sl] = v * keep
                rowsb[slot, n, sl] = bias_v[sl]

    def body(h, carry):
        for p in range(_DEPTH):
            g = _DEPTH * h + p
            outb, semo = (outb0, semo0) if p % 2 == 0 else (outb1, semo1)
            wait_gathers(g, p)

            @pl.when(g >= 2)
            def _():
                out_copy(g - 2, outb, semo).wait()

            compute(g, p, outb)
            out_copy(g, outb, semo).start()
            nxt = g + _DEPTH

            @pl.when(nxt < nreal)
            def _():
                start_gathers(nxt, p)
        return carry

    lax.fori_loop(0, nreal // _DEPTH, body, 0)
    out_copy(nreal - 2, outb0, semo0).wait()
    out_copy(nreal - 1, outb1, semo1).wait()


def kernel(x, spiral_adj, W, b):
    B, N, F = x.shape
    S = spiral_adj.shape[-1]
    O = W.shape[0]
    assert (B, N, F, S, O) == (1, _N, _F, _S, _O)

    x2 = x.reshape(N, F)
    wt = jnp.transpose(W.reshape(O, S, F), (2, 1, 0)).reshape(F, S * O)
    ys = _tc_matmul(x2, wt)

    idxT = jnp.pad(spiral_adj[0].astype(jnp.int32).T, ((0, 0), (0, _NPAD - N)))
    idx2 = idxT.reshape(-1)

    out = _sc_gather_reduce(*ys, idx2, b)
    return out.reshape(B, N, O)
